# baseline (speedup 1.0000x reference)
import functools

import jax
import jax.numpy as jnp
from jax import lax
from jax.experimental import pallas as pl
from jax.experimental.pallas import tpu as pltpu
from jax.experimental.pallas import tpu_sc as plsc

B = 16384
D = 16
L = 16
NC = 2
NS = 16
NW = NC * NS
BPW = B // NW
TL = 128
NUM = 1000000
NCOL = (NUM + TL - 1) // TL
CPW = (NCOL + NW - 1) // NW
CHUNK = 16
CW = CHUNK * TL
NCHUNK = (CPW + CHUNK - 1) // CHUNK
CAP = 2048
CCAP = 2048
NVEC = B // L
STG = (B + L) * D


def _phase1_body(ui_hbm, ii_hbm, ue_hbm, ie_hbm,
                 ustg_hbm, istg_hbm,
                 uidx_v, iidx_v, uchunk, ichunk,
                 uval, upos, ival, ipos, cval, cpos, ebuf,
                 ucsem, icsem, ssem):
    wid = lax.axis_index("s") * NC + lax.axis_index("c")
    col_lo = wid * CPW
    vlo = col_lo * TL
    vhi = jnp.minimum((col_lo + CPW) * TL, NUM)

    lane = lax.iota(jnp.int32, L)

    def fetch(c, tab_hbm, buf, sem):
        c0 = pl.multiple_of(
            jnp.minimum(col_lo + c * CHUNK, NCOL - CHUNK) * TL, TL)
        pltpu.async_copy(tab_hbm.at[:, pl.ds(c0, CW)], buf, sem)

    def drain(tab_hbm, buf, sem):
        pltpu.make_async_copy(tab_hbm.at[:, pl.ds(0, CW)], buf, sem).wait()

    fetch(0, ue_hbm, uchunk, ucsem)
    fetch(0, ie_hbm, ichunk, icsem)

    pltpu.sync_copy(ui_hbm, uidx_v)
    pltpu.sync_copy(ii_hbm, iidx_v)

    def scan_from(idx_ref, val_ref, pos_ref, start):
        def body(k, carry):
            cnt, nxt = carry
            s = k * L
            vv = idx_ref[pl.ds(s, L)]
            m = (vv >= vlo) & (vv < vhi)
            npop = plsc.all_reduce_population_count(m)[0]
            accept = (k >= start) & (cnt + npop <= CAP) & (nxt == NVEC)

            @pl.when(accept & (npop > 0))
            def _():
                plsc.store_compressed(val_ref.at[pl.ds(cnt, L)], vv, mask=m)
                plsc.store_compressed(pos_ref.at[pl.ds(cnt, L)], s + lane,
                                      mask=m)
            cnt = jnp.where(accept, cnt + npop, cnt)
            rejected = (k >= start) & (nxt == NVEC) & jnp.logical_not(accept)
            nxt = jnp.where(rejected, k, nxt)
            return cnt, nxt

        return lax.fori_loop(0, NVEC, body,
                             (jnp.int32(0), jnp.int32(NVEC)))

    def process(c, buf, val_ref, pos_ref, cnt, stg):
        a_lo = (col_lo + c * CHUNK) * TL
        a_hi = jnp.minimum(a_lo + CW, vhi)
        base = jnp.minimum(col_lo + c * CHUNK, NCOL - CHUNK) * TL

        def compact(k, cc):
            s = k * L
            vv = val_ref[pl.ds(s, L)]
            pp = pos_ref[pl.ds(s, L)]
            m = (vv >= a_lo) & (vv < a_hi) & ((s + lane) < cnt)

            @pl.when(plsc.all_reduce_population_count(m)[0] > 0)
            def _():
                plsc.store_compressed(cval.at[pl.ds(cc, L)], vv, mask=m)
                plsc.store_compressed(cpos.at[pl.ds(cc, L)], pp, mask=m)
            return cc + plsc.all_reduce_population_count(m)[0]

        ccnt = lax.fori_loop(0, (cnt + L - 1) // L, compact, jnp.int32(0))

        def batch(b, _):
            s = b * L
            vv = cval[pl.ds(s, L)]
            pp = cpos[pl.ds(s, L)]
            valid = (s + lane) < ccnt
            cc = jnp.where(valid, vv - base, 0)
            pv = jnp.where(valid, pp, B)
            for j in range(L):
                col = jnp.full((L,), 0, jnp.int32) + cc[j]
                ebuf[pl.ds(j * D, D)] = plsc.load_gather(buf, [lane, col])
            for j in range(L):
                pltpu.async_copy(ebuf.at[pl.ds(j * D, D)],
                                 stg.at[pv[j] * D + lane], ssem)
            pltpu.make_async_copy(ue_hbm.at[0, pl.ds(0, L * D)], ebuf,
                                  ssem).wait()
            return 0

        lax.fori_loop(0, (ccnt + L - 1) // L, batch, 0)

    def round_body(carry):
        ustart, istart = carry
        ucnt, unext = scan_from(uidx_v, uval, upos, ustart)
        icnt, inext = scan_from(iidx_v, ival, ipos, istart)

        def step(c, _):
            drain(ue_hbm, uchunk, ucsem)
            process(c, uchunk, uval, upos, ucnt, ustg_hbm)

            @pl.when(c + 1 < NCHUNK)
            def _():
                fetch(c + 1, ue_hbm, uchunk, ucsem)
            drain(ie_hbm, ichunk, icsem)
            process(c, ichunk, ival, ipos, icnt, istg_hbm)

            @pl.when(c + 1 < NCHUNK)
            def _():
                fetch(c + 1, ie_hbm, ichunk, icsem)
            return 0

        lax.fori_loop(0, NCHUNK, step, 0)

        @pl.when((unext < NVEC) | (inext < NVEC))
        def _():
            fetch(0, ue_hbm, uchunk, ucsem)
            fetch(0, ie_hbm, ichunk, icsem)
        return unext, inext

    lax.while_loop(lambda c: (c[0] < NVEC) | (c[1] < NVEC), round_body,
                   (jnp.int32(0), jnp.int32(0)))


def _phase2_body(ustg_hbm, istg_hbm, out_hbm, us_v, is_v, prod_v, out_v, sem):
    wid = lax.axis_index("s") * NC + lax.axis_index("c")
    base = wid * BPW

    pltpu.sync_copy(ustg_hbm.at[pl.ds(base * D, BPW * D)], us_v)
    pltpu.sync_copy(istg_hbm.at[pl.ds(base * D, BPW * D)], is_v)

    lane16 = lax.iota(jnp.int32, L) * D

    def block(b, _):
        s = b * L
        for k in range(L):
            prod_v[pl.ds(k * D, D)] = (us_v[pl.ds((s + k) * D, D)]
                                       * is_v[pl.ds((s + k) * D, D)])
        acc = plsc.load_gather(prod_v, [lane16])
        for d in range(1, D):
            acc = acc + plsc.load_gather(prod_v, [lane16 + d])
        out_v[pl.ds(s, L)] = acc
        return 0

    lax.fori_loop(0, BPW // L, block, 0)
    pltpu.sync_copy(out_v, out_hbm.at[pl.ds(base, BPW)])


_SC_PARAMS = pltpu.CompilerParams(
    needs_layout_passes=False, use_tc_tiling_on_sc=True
)
_MESH = plsc.VectorSubcoreMesh(core_axis_name="c", subcore_axis_name="s")


@functools.partial(
    pl.kernel,
    out_type=(jax.ShapeDtypeStruct((STG,), jnp.float32),
              jax.ShapeDtypeStruct((STG,), jnp.float32)),
    mesh=_MESH,
    compiler_params=_SC_PARAMS,
    scratch_types=[
        pltpu.VMEM((B,), jnp.int32),
        pltpu.VMEM((B,), jnp.int32),
        pltpu.VMEM((D, CW), jnp.float32),
        pltpu.VMEM((D, CW), jnp.float32),
        pltpu.VMEM((CAP + L,), jnp.int32),
        pltpu.VMEM((CAP + L,), jnp.int32),
        pltpu.VMEM((CAP + L,), jnp.int32),
        pltpu.VMEM((CAP + L,), jnp.int32),
        pltpu.VMEM((CCAP + L,), jnp.int32),
        pltpu.VMEM((CCAP + L,), jnp.int32),
        pltpu.VMEM((L * D,), jnp.float32),
        pltpu.SemaphoreType.DMA,
        pltpu.SemaphoreType.DMA,
        pltpu.SemaphoreType.DMA,
    ],
)
def _phase1(*refs):
    _phase1_body(*refs)


@functools.partial(
    pl.kernel,
    out_type=jax.ShapeDtypeStruct((B,), jnp.float32),
    mesh=_MESH,
    compiler_params=_SC_PARAMS,
    scratch_types=[
        pltpu.VMEM((BPW * D,), jnp.float32),
        pltpu.VMEM((BPW * D,), jnp.float32),
        pltpu.VMEM((L * D,), jnp.float32),
        pltpu.VMEM((BPW,), jnp.float32),
        pltpu.SemaphoreType.DMA,
    ],
)
def _phase2(*refs):
    _phase2_body(*refs)


def kernel(user_indices, item_indices, user_embedding, item_embedding):
    ustg, istg = _phase1(user_indices, item_indices,
                         user_embedding.T, item_embedding.T)
    return _phase2(ustg, istg)


# speedup vs baseline: 66.1813x; 66.1813x over previous
import functools

import jax
import jax.numpy as jnp
from jax import lax
from jax.experimental import pallas as pl
from jax.experimental.pallas import tpu as pltpu
from jax.experimental.pallas import tpu_sc as plsc

B = 16384
D = 16
L = 16
NC = 2
NS = 16
NW = NC * NS
BPW = B // NW
TL = 128
G = 8
RING = G * TL


def _mf_body(ui_hbm, ii_hbm, ue_hbm, ie_hbm, out_hbm,
             uidx_v, iidx_v, ua, ub, ia, ib, prod_v, out_v,
             uasem, ubsem, iasem, ibsem):
    wid = lax.axis_index("s") * NC + lax.axis_index("c")
    base = wid * BPW

    pltpu.sync_copy(ui_hbm.at[pl.ds(base, BPW)], uidx_v)
    pltpu.sync_copy(ii_hbm.at[pl.ds(base, BPW)], iidx_v)

    lane = lax.iota(jnp.int32, L)
    lane16 = lane * D

    def issue_half(uvec, ivec, lo, uring, iring, us, isem_):
        for j in range(G):
            uc = pl.multiple_of((uvec[lo + j] // TL) * TL, TL)
            ic = pl.multiple_of((ivec[lo + j] // TL) * TL, TL)
            pltpu.async_copy(ue_hbm.at[:, pl.ds(uc, TL)],
                             uring.at[:, pl.ds(j * TL, TL)], us)
            pltpu.async_copy(ie_hbm.at[:, pl.ds(ic, TL)],
                             iring.at[:, pl.ds(j * TL, TL)], isem_)

    def drain_half(uring, iring, us, isem_):
        pltpu.make_async_copy(ue_hbm.at[:, pl.ds(0, RING)], uring, us).wait()
        pltpu.make_async_copy(ie_hbm.at[:, pl.ds(0, RING)], iring, isem_).wait()

    def process_half(uvec, ivec, lo, uring, iring, pbase):
        for j in range(G):
            ucc = jnp.full((L,), j * TL, jnp.int32) + uvec[lo + j] % TL
            icc = jnp.full((L,), j * TL, jnp.int32) + ivec[lo + j] % TL
            ucol = plsc.load_gather(uring, [lane, ucc])
            icol = plsc.load_gather(iring, [lane, icc])
            prod_v[pl.ds(pbase + j * D, D)] = ucol * icol

    def block(p, _):
        s = p * L
        uvec = uidx_v[pl.ds(s, L)]
        ivec = iidx_v[pl.ds(s, L)]
        issue_half(uvec, ivec, 0, ua, ia, uasem, iasem)
        issue_half(uvec, ivec, G, ub, ib, ubsem, ibsem)
        drain_half(ua, ia, uasem, iasem)
        process_half(uvec, ivec, 0, ua, ia, 0)
        drain_half(ub, ib, ubsem, ibsem)
        process_half(uvec, ivec, G, ub, ib, G * D)
        acc = plsc.load_gather(prod_v, [lane16])
        for d in range(1, D):
            acc = acc + plsc.load_gather(prod_v, [lane16 + d])
        out_v[pl.ds(s, L)] = acc
        return 0

    lax.fori_loop(0, BPW // L, block, 0)

    pltpu.sync_copy(out_v, out_hbm.at[pl.ds(base, BPW)])


@functools.partial(
    pl.kernel,
    out_type=jax.ShapeDtypeStruct((B,), jnp.float32),
    mesh=plsc.VectorSubcoreMesh(core_axis_name="c", subcore_axis_name="s"),
    compiler_params=pltpu.CompilerParams(
        needs_layout_passes=False, use_tc_tiling_on_sc=True
    ),
    scratch_types=[
        pltpu.VMEM((BPW,), jnp.int32),
        pltpu.VMEM((BPW,), jnp.int32),
        pltpu.VMEM((D, RING), jnp.float32),
        pltpu.VMEM((D, RING), jnp.float32),
        pltpu.VMEM((D, RING), jnp.float32),
        pltpu.VMEM((D, RING), jnp.float32),
        pltpu.VMEM((L * D,), jnp.float32),
        pltpu.VMEM((BPW,), jnp.float32),
        pltpu.SemaphoreType.DMA,
        pltpu.SemaphoreType.DMA,
        pltpu.SemaphoreType.DMA,
        pltpu.SemaphoreType.DMA,
    ],
)
def _mf_kernel(*refs):
    _mf_body(*refs)


def kernel(user_indices, item_indices, user_embedding, item_embedding):
    return _mf_kernel(user_indices, item_indices,
                      user_embedding.T, item_embedding.T)
